# baseline (speedup 1.0000x reference)
import jax
import jax.numpy as jnp
from jax.experimental import pallas as pl
from jax.experimental.pallas import tpu as pltpu

NEG = -3e38
ROWS = 8


def _shift_right(x, d, fill):
    rolled = jnp.roll(x, d, axis=-1)
    lane = jax.lax.broadcasted_iota(jnp.int32, x.shape, x.ndim - 1)
    return jnp.where(lane < d, fill, rolled)


def _cumsum_j(x):
    for d in (1, 2, 4, 8, 16, 32, 64):
        x = x + _shift_right(x, d, 0.0)
    return x


def _cummax_j(x):
    for d in (1, 2, 4, 8, 16, 32, 64):
        x = jnp.maximum(x, _shift_right(x, d, NEG))
    return x


def _row_update(row, half_a, b, M):
    half_b = 0.5 * b
    S = jax.lax.dot_general(
        half_b, M, (((1,), (0,)), ((), ())),
        preferred_element_type=jnp.float32,
    )
    tmp = row + half_a
    cand_up = tmp + half_b
    cand_diag = _shift_right(tmp, 1, NEG) + half_b
    base = jnp.maximum(cand_up, cand_diag)
    return S + _cummax_j(base - S), half_b


def _dp_kernel(img_ref, m_ref, out_ref, row_ref, prev_ref):
    R, Bb, J = img_ref.shape
    t = pl.program_id(1)
    M = m_ref[:, :]

    @pl.when(t == 0)
    def _init():
        r0 = img_ref[0]
        half_r0 = 0.5 * r0
        S0 = jax.lax.dot_general(
            half_r0, M, (((1,), (0,)), ((), ())),
            preferred_element_type=jnp.float32,
        )
        row = S0 + (r0[:, 0:1] - S0[:, 0:1])
        half_a = half_r0
        for r in range(1, R):
            row, half_a = _row_update(row, half_a, img_ref[r], M)
        row_ref[:, :] = row
        prev_ref[:, :] = half_a

    @pl.when(t != 0)
    def _step():
        row = row_ref[:, :]
        half_a = prev_ref[:, :]
        for r in range(R):
            row, half_a = _row_update(row, half_a, img_ref[r], M)
        row_ref[:, :] = row
        prev_ref[:, :] = half_a

    out_ref[:, :] = row_ref[:, J - 1 : J]


@jax.jit
def kernel(images):
    B, I, J = images.shape
    Bb = 512
    nchunk = B // Bb
    k = jnp.arange(J)
    M = (2.0 * (k[:, None] < k[None, :]) + (k[:, None] == k[None, :])).astype(
        jnp.float32
    )
    outs = []
    for c in range(nchunk):
        chunk = jax.lax.slice_in_dim(images, c * Bb, (c + 1) * Bb, axis=0)
        imgs_t = jnp.swapaxes(chunk, 0, 1)
        out = pl.pallas_call(
            _dp_kernel,
            grid=(1, I // ROWS),
            in_specs=[
                pl.BlockSpec((ROWS, Bb, J), lambda b, t: (t, b, 0)),
                pl.BlockSpec((J, J), lambda b, t: (0, 0)),
            ],
            out_specs=pl.BlockSpec((Bb, 1), lambda b, t: (b, 0)),
            out_shape=jax.ShapeDtypeStruct((Bb, 1), jnp.float32),
            scratch_shapes=[
                pltpu.VMEM((Bb, J), jnp.float32),
                pltpu.VMEM((Bb, J), jnp.float32),
            ],
            compiler_params=pltpu.CompilerParams(
                dimension_semantics=("arbitrary", "arbitrary"),
            ),
        )(imgs_t, M)
        outs.append(out[:, 0])
    return jnp.concatenate(outs)


# speedup vs baseline: 1.1271x; 1.1271x over previous
import jax
import jax.numpy as jnp
from jax.experimental import pallas as pl
from jax.experimental.pallas import tpu as pltpu

NEG = -3e38
ROWS = 8


def _shift_right(x, d, fill):
    rolled = jnp.roll(x, d, axis=-1)
    lane = jax.lax.broadcasted_iota(jnp.int32, x.shape, x.ndim - 1)
    return jnp.where(lane < d, fill, rolled)


def _cummax_j(x):
    for d in (1, 2, 4, 8, 16, 32, 64):
        x = jnp.maximum(x, _shift_right(x, d, NEG))
    return x


def _row_update(row, half_a, b, M):
    half_b = 0.5 * b
    S = jax.lax.dot_general(
        half_b, M, (((1,), (0,)), ((), ())),
        preferred_element_type=jnp.float32,
    )
    tmp = row + half_a
    cand_up = tmp + half_b
    cand_diag = _shift_right(tmp, 1, NEG) + half_b
    base = jnp.maximum(cand_up, cand_diag)
    return S + _cummax_j(base - S), half_b


def _dp_kernel(img_ref, m_ref, out_ref, row_ref, prev_ref):
    Bb, RJ = img_ref.shape
    J = m_ref.shape[0]
    R = RJ // J
    t = pl.program_id(1)
    M = m_ref[:, :]

    def img_row(r):
        return img_ref[:, r * J : (r + 1) * J]

    @pl.when(t == 0)
    def _init():
        r0 = img_row(0)
        half_r0 = 0.5 * r0
        S0 = jax.lax.dot_general(
            half_r0, M, (((1,), (0,)), ((), ())),
            preferred_element_type=jnp.float32,
        )
        row = S0 + (r0[:, 0:1] - S0[:, 0:1])
        half_a = half_r0
        for r in range(1, R):
            row, half_a = _row_update(row, half_a, img_row(r), M)
        row_ref[:, :] = row
        prev_ref[:, :] = half_a

    @pl.when(t != 0)
    def _step():
        row = row_ref[:, :]
        half_a = prev_ref[:, :]
        for r in range(R):
            row, half_a = _row_update(row, half_a, img_row(r), M)
        row_ref[:, :] = row
        prev_ref[:, :] = half_a

    out_ref[:, :] = row_ref[:, J - 1 : J]


@jax.jit
def kernel(images):
    B, I, J = images.shape
    Bb = 512
    nb = B // Bb
    grid = (nb, I // ROWS)
    flat = images.reshape(B, I * J)
    k = jnp.arange(J)
    M = (2.0 * (k[:, None] < k[None, :]) + (k[:, None] == k[None, :])).astype(
        jnp.float32
    )
    out = pl.pallas_call(
        _dp_kernel,
        grid=grid,
        in_specs=[
            pl.BlockSpec((Bb, ROWS * J), lambda b, t: (b, t)),
            pl.BlockSpec((J, J), lambda b, t: (0, 0)),
        ],
        out_specs=pl.BlockSpec((Bb, 1), lambda b, t: (b, 0)),
        out_shape=jax.ShapeDtypeStruct((B, 1), jnp.float32),
        scratch_shapes=[
            pltpu.VMEM((Bb, J), jnp.float32),
            pltpu.VMEM((Bb, J), jnp.float32),
        ],
        compiler_params=pltpu.CompilerParams(
            dimension_semantics=("arbitrary", "arbitrary"),
        ),
    )(flat, M)
    return out[:, 0]


# speedup vs baseline: 1.5434x; 1.3693x over previous
import jax
import jax.numpy as jnp
from jax.experimental import pallas as pl
from jax.experimental.pallas import tpu as pltpu

NEG = -3e38
ROWS = 8


def _shift_right(x, d, fill):
    rolled = jnp.roll(x, d, axis=-1)
    lane = jax.lax.broadcasted_iota(jnp.int32, x.shape, x.ndim - 1)
    return jnp.where(lane < d, fill, rolled)


def _cummax_j(x):
    for d in (1, 2, 4, 8, 16, 32, 64):
        x = jnp.maximum(x, _shift_right(x, d, NEG))
    return x


def _row_update(row, half_a, b, M):
    half_b = 0.5 * b
    S = jax.lax.dot_general(
        half_b, M, (((1,), (0,)), ((), ())),
        preferred_element_type=jnp.float32,
    )
    tmp = row + half_a
    base = jnp.maximum(tmp, _shift_right(tmp, 1, NEG)) + half_b
    return S + _cummax_j(base - S), half_b


def _dp_kernel(nb, nt, img_ref, m_ref, out_ref, row_ref, prev_ref, buf_ref, sem):
    B, I, J = img_ref.shape
    Bb = row_ref.shape[0]
    b = pl.program_id(0)
    t = pl.program_id(1)
    M = m_ref[:, :]

    def strip_copies(bi, ti, slot):
        return [
            pltpu.make_async_copy(
                img_ref.at[pl.ds(bi * Bb, Bb), ti * ROWS + r, :],
                buf_ref.at[slot, :, pl.ds(r * J, J)],
                sem.at[slot, r],
            )
            for r in range(ROWS)
        ]

    @pl.when(jnp.logical_and(b == 0, t == 0))
    def _start_first():
        for c in strip_copies(0, 0, 0):
            c.start()

    nxt = t + 1
    nb_i = jnp.where(nxt == nt, b + 1, b)
    nt_i = jnp.where(nxt == nt, 0, nxt)

    @pl.when(nb_i < nb)
    def _prefetch():
        for c in strip_copies(nb_i, nt_i, nxt % 2):
            c.start()

    slot = t % 2
    for c in strip_copies(b, t, slot):
        c.wait()

    def img_row(r):
        return buf_ref[slot, :, r * J : (r + 1) * J]

    @pl.when(t == 0)
    def _init():
        r0 = img_row(0)
        half_r0 = 0.5 * r0
        S0 = jax.lax.dot_general(
            half_r0, M, (((1,), (0,)), ((), ())),
            preferred_element_type=jnp.float32,
        )
        row = S0 + (r0[:, 0:1] - S0[:, 0:1])
        half_a = half_r0
        for r in range(1, ROWS):
            row, half_a = _row_update(row, half_a, img_row(r), M)
        row_ref[:, :] = row
        prev_ref[:, :] = half_a

    @pl.when(t != 0)
    def _step():
        row = row_ref[:, :]
        half_a = prev_ref[:, :]
        for r in range(ROWS):
            row, half_a = _row_update(row, half_a, img_row(r), M)
        row_ref[:, :] = row
        prev_ref[:, :] = half_a

    out_ref[:, :] = row_ref[:, J - 1 : J]


@jax.jit
def kernel(images):
    import functools

    B, I, J = images.shape
    Bb = 512
    nb = B // Bb
    nt = I // ROWS
    k = jnp.arange(J)
    M = (2.0 * (k[:, None] < k[None, :]) + (k[:, None] == k[None, :])).astype(
        jnp.float32
    )
    out = pl.pallas_call(
        functools.partial(_dp_kernel, nb, nt),
        grid=(nb, nt),
        in_specs=[
            pl.BlockSpec(memory_space=pl.ANY),
            pl.BlockSpec((J, J), lambda b, t: (0, 0)),
        ],
        out_specs=pl.BlockSpec((Bb, 1), lambda b, t: (b, 0)),
        out_shape=jax.ShapeDtypeStruct((B, 1), jnp.float32),
        scratch_shapes=[
            pltpu.VMEM((Bb, J), jnp.float32),
            pltpu.VMEM((Bb, J), jnp.float32),
            pltpu.VMEM((2, Bb, ROWS * J), jnp.float32),
            pltpu.SemaphoreType.DMA((2, ROWS)),
        ],
        compiler_params=pltpu.CompilerParams(
            dimension_semantics=("arbitrary", "arbitrary"),
        ),
    )(images, M)
    return out[:, 0]
